# baseline (speedup 1.0000x reference)
import functools

import jax
import jax.numpy as jnp
from jax import lax
from jax.experimental import pallas as pl
from jax.experimental.pallas import tpu as pltpu
from jax.experimental.pallas import tpu_sc as plsc

NOBS = 16384
CH = 10
NGRID = 100000
NCOLS = 33
L = 16
TW = 128
TROWS = (NGRID * NCOLS + TW - 1) // TW

_info = plsc.get_sparse_core_info()
NC = _info.num_cores
NS = _info.num_subcores
NW = NC * NS
BPW = NOBS // NW
OCH = 128
NCH = BPW // OCH
GC = 128

_mesh = plsc.VectorSubcoreMesh(core_axis_name="c", subcore_axis_name="s")

_f32 = jnp.float32
_i32 = jnp.int32


@functools.partial(
    pl.kernel,
    mesh=_mesh,
    compiler_params=pltpu.CompilerParams(needs_layout_passes=False),
    out_type=jax.ShapeDtypeStruct((NOBS,), _f32),
    scratch_types=[
        pltpu.VMEM((2 * BPW,), _i32),
        pltpu.VMEM((BPW,), _i32),
        [pltpu.VMEM((2 * OCH, TW), _f32) for _ in range(2)],
        pltpu.VMEM((BPW,), _f32),
        [pltpu.SemaphoreType.DMA for _ in range(2)],
        pltpu.SemaphoreType.DMA,
    ],
)
def _seaice_sc(idx_hbm, d_hbm, tab_hbm, s_hbm,
               idx_v, d_v, bufs, s_v, sem_g, sem_s):
    wid = lax.axis_index("s") * NC + lax.axis_index("c")
    base = wid * BPW

    pltpu.sync_copy(idx_hbm.at[pl.ds(2 * base, 2 * BPW)], idx_v)
    pltpu.sync_copy(d_hbm.at[pl.ds(base, BPW)], d_v)

    def fire(ch):
        b = ch % 2
        return [
            pltpu.async_copy(
                tab_hbm.at[idx_v.at[pl.ds(2 * ch * OCH + t * GC, GC)]],
                bufs[b].at[pl.ds(t * GC, GC), :], sem_g[b])
            for t in range(2 * OCH // GC)
        ]

    lane = lax.iota(_i32, L)
    a0 = _f32(0.2)
    a1 = _f32(0.3)
    a2 = _f32(0.5)

    cps = fire(0)
    for ch in range(NCH):
        nxt = fire(ch + 1) if ch + 1 < NCH else None
        for cp in cps:
            cp.wait()
        buf = bufs[ch % 2]

        def s_body(j, carry, ch=ch, buf=buf):
            o16 = j * L + lane
            off = ch * OCH + j * L
            dd = d_v[pl.ds(off, L)]
            rb = 2 * o16
            g0 = plsc.load_gather(buf, [rb + lax.shift_right_logical(dd, 7),
                                        jnp.bitwise_and(dd, TW - 1)])
            e1 = dd + 1
            g1 = plsc.load_gather(buf, [rb + lax.shift_right_logical(e1, 7),
                                        jnp.bitwise_and(e1, TW - 1)])
            e2 = dd + 2
            g2 = plsc.load_gather(buf, [rb + lax.shift_right_logical(e2, 7),
                                        jnp.bitwise_and(e2, TW - 1)])
            s_v[pl.ds(off, L)] = a0 * g0 + a1 * g1 + a2 * g2
            return carry

        lax.fori_loop(0, OCH // L, s_body, 0)
        cps = nxt

    pltpu.async_copy(s_v, s_hbm.at[pl.ds(base, BPW)], sem_s).wait()


_BLKC = 2048


def _blend_body(s_ref, eo_ref, es_ref, o_ref):
    sv = s_ref[...][None, :]
    eo = eo_ref[...]
    es = es_ref[...]
    o_ref[...] = eo + sv * (es - eo)


def _blend_tc(s, eo_t, es_t):
    return pl.pallas_call(
        _blend_body,
        grid=(NOBS // _BLKC,),
        in_specs=[
            pl.BlockSpec((_BLKC,), lambda i: (i,)),
            pl.BlockSpec((CH, _BLKC), lambda i: (0, i)),
            pl.BlockSpec((CH, _BLKC), lambda i: (0, i)),
        ],
        out_specs=pl.BlockSpec((CH, _BLKC), lambda i: (0, i)),
        out_shape=jax.ShapeDtypeStruct((CH, NOBS), _f32),
    )(s, eo_t, es_t)


def kernel(geolocation, emis_ocean, emis_seaice, tsfc, seaice, seaice_background):
    del tsfc, seaice_background
    rows = geolocation[:, 0]
    cols = geolocation[:, 1]
    f = rows * NCOLS + cols
    q = lax.shift_right_logical(f, 7)
    d = jnp.bitwise_and(f, TW - 1)
    idx2 = jnp.stack([q, q + 1], axis=1).reshape(2 * NOBS)
    flat = seaice.reshape(NGRID * NCOLS)
    tab2 = jnp.pad(flat, (0, TROWS * TW - NGRID * NCOLS)).reshape(TROWS, TW)
    s = _seaice_sc(idx2, d, tab2)
    out_t = _blend_tc(s, emis_ocean.T, emis_seaice.T)
    return (out_t.T, s)


# speedup vs baseline: 1.7605x; 1.7605x over previous
import functools

import jax
import jax.numpy as jnp
from jax import lax
from jax.experimental import pallas as pl
from jax.experimental.pallas import tpu as pltpu
from jax.experimental.pallas import tpu_sc as plsc

NOBS = 16384
CH = 10
NCOLS = 33
L = 16

_info = plsc.get_sparse_core_info()
NC = _info.num_cores
NS = _info.num_subcores
NW = NC * NS
BPW = NOBS // NW
RCH = 64
NWAVE = BPW // RCH

_mesh = plsc.VectorSubcoreMesh(core_axis_name="c", subcore_axis_name="s")

_f32 = jnp.float32
_i32 = jnp.int32


@functools.partial(
    pl.kernel,
    mesh=_mesh,
    compiler_params=pltpu.CompilerParams(
        needs_layout_passes=False, use_tc_tiling_on_sc=True),
    out_type=jax.ShapeDtypeStruct((NOBS,), _f32),
    scratch_types=[
        pltpu.VMEM((BPW,), _i32),
        pltpu.VMEM((BPW,), _i32),
        pltpu.VMEM((BPW,), _f32),
        [pltpu.VMEM((RCH, NCOLS), _f32) for _ in range(2)],
        [pltpu.SemaphoreType.DMA for _ in range(2)],
        pltpu.SemaphoreType.DMA,
    ],
)
def _seaice_sc(rows_hbm, cols_hbm, tab_hbm, s_hbm,
               row_v, col_v, s_v, rows_b, sem_rows, sem_s):
    wid = lax.axis_index("s") * NC + lax.axis_index("c")
    base = wid * BPW

    pltpu.sync_copy(rows_hbm.at[pl.ds(base, BPW)], row_v)
    pltpu.sync_copy(cols_hbm.at[pl.ds(base, BPW)], col_v)

    def fire_wave(w):
        b = w % 2
        buf = rows_b[b]

        def body(j, carry):
            rv = row_v[pl.ds(w * RCH + j * L, L)]
            for k in range(L):
                pltpu.async_copy(tab_hbm.at[pl.ds(rv[k], 1), :],
                                 buf.at[pl.ds(j * L + k, 1), :],
                                 sem_rows[b])
            return carry

        lax.fori_loop(0, RCH // L, body, 0)

    def drain_wave(w):
        b = w % 2
        pltpu.make_async_copy(
            tab_hbm.at[pl.ds(0, RCH), :], rows_b[b], sem_rows[b]).wait()

    lane = lax.iota(_i32, L)
    a0 = _f32(0.2)
    a1 = _f32(0.3)
    a2 = _f32(0.5)

    fire_wave(0)

    for w in range(NWAVE):
        if w + 1 < NWAVE:
            fire_wave(w + 1)

        drain_wave(w)
        buf = rows_b[w % 2]

        def s_body(j, carry, w=w, buf=buf):
            i16 = j * L + lane
            off = w * RCH + j * L
            c = col_v[pl.ds(off, L)]
            g0 = plsc.load_gather(buf, [i16, c])
            g1 = plsc.load_gather(buf, [i16, c + 1])
            g2 = plsc.load_gather(buf, [i16, c + 2])
            s_v[pl.ds(off, L)] = a0 * g0 + a1 * g1 + a2 * g2
            return carry

        lax.fori_loop(0, RCH // L, s_body, 0)

    pltpu.async_copy(s_v, s_hbm.at[pl.ds(base, BPW)], sem_s).wait()


_BLKC = 2048


def _blend_body(s_ref, eo_ref, es_ref, o_ref):
    sv = s_ref[...][None, :]
    eo = eo_ref[...]
    es = es_ref[...]
    o_ref[...] = eo + sv * (es - eo)


def _blend_tc(s, eo_t, es_t):
    return pl.pallas_call(
        _blend_body,
        grid=(NOBS // _BLKC,),
        in_specs=[
            pl.BlockSpec((_BLKC,), lambda i: (i,)),
            pl.BlockSpec((CH, _BLKC), lambda i: (0, i)),
            pl.BlockSpec((CH, _BLKC), lambda i: (0, i)),
        ],
        out_specs=pl.BlockSpec((CH, _BLKC), lambda i: (0, i)),
        out_shape=jax.ShapeDtypeStruct((CH, NOBS), _f32),
    )(s, eo_t, es_t)


def kernel(geolocation, emis_ocean, emis_seaice, tsfc, seaice, seaice_background):
    del tsfc, seaice_background
    s = _seaice_sc(geolocation[:, 0], geolocation[:, 1], seaice)
    out_t = _blend_tc(s, emis_ocean.T, emis_seaice.T)
    return (out_t.T, s)
